# baseline (speedup 1.0000x reference)
import functools

import jax
import jax.numpy as jnp
from jax import lax
from jax.experimental import pallas as pl
from jax.experimental.pallas import tpu as pltpu
from jax.experimental.pallas import tpu_sc as plsc

MAX_T = 8192
DM = 1024
NB = 4

NC = 2
NS = 16
NW = NC * NS

TT = 16
T_PER_W = MAX_T // NW
N_STEPS = T_PER_W // TT
CHUNK = TT * DM
LANES = 16


def _sc_add(x_hbm, emb_hbm, out_hbm, emb_v, x_v):
    wid = lax.axis_index("s") * NC + lax.axis_index("c")
    base = wid * T_PER_W

    def step(s, _):
        ts = (base + s * TT) * DM
        pltpu.sync_copy(emb_hbm.at[pl.ds(ts, CHUNK)], emb_v)

        def one_batch(b, _):
            xo = b * (MAX_T * DM) + ts
            pltpu.sync_copy(x_hbm.at[pl.ds(xo, CHUNK)], x_v)

            def chunk_add(j, _):
                sl = pl.ds(j * LANES, LANES)
                x_v[sl] = x_v[sl] + emb_v[sl]
                return 0

            lax.fori_loop(0, CHUNK // LANES, chunk_add, 0)
            pltpu.sync_copy(x_v, out_hbm.at[pl.ds(xo, CHUNK)])
            return 0

        lax.fori_loop(0, NB, one_batch, 0)
        return 0

    lax.fori_loop(0, N_STEPS, step, 0)


@jax.jit
def _sc_kernel(x_flat, emb_flat):
    mesh = plsc.VectorSubcoreMesh(core_axis_name="c", subcore_axis_name="s")
    return pl.kernel(
        _sc_add,
        mesh=mesh,
        out_type=jax.ShapeDtypeStruct((NB * MAX_T * DM,), jnp.float32),
        scratch_types=[
            pltpu.VMEM((CHUNK,), jnp.float32),
            pltpu.VMEM((CHUNK,), jnp.float32),
        ],
    )(x_flat, emb_flat)


def kernel(x, emb):
    b, t, d = x.shape
    out = _sc_kernel(x.reshape(-1), emb.reshape(-1))
    return out.reshape(b, t, d)


# speedup vs baseline: 1.5634x; 1.5634x over previous
import jax
import jax.numpy as jnp
from jax import lax
from jax.experimental import pallas as pl
from jax.experimental.pallas import tpu as pltpu
from jax.experimental.pallas import tpu_sc as plsc

MAX_T = 8192
DM = 1024
NB = 4

NC = 2
NS = 16
NW = NC * NS

TT = 8
T_PER_W = MAX_T // NW
N_STEPS = T_PER_W // TT
CHUNK = TT * DM
LANES = 16
UNROLL = 4
N_VEC = CHUNK // LANES


def _sc_add(x_hbm, emb_hbm, out_hbm, emb_v, x_v,
            sem_e0, sem_e1, sem_i0, sem_i1, sem_i2,
            sem_o0, sem_o1, sem_o2):
    wid = lax.axis_index("s") * NC + lax.axis_index("c")
    base = wid * T_PER_W

    sem_e = (sem_e0, sem_e1)
    sem_i = (sem_i0, sem_i1, sem_i2)
    sem_o = (sem_o0, sem_o1, sem_o2)

    def start_in(s):
        p, ep = s % 3, s % 2
        ts = (base + s * TT) * DM
        he = pltpu.async_copy(emb_hbm.at[pl.ds(ts, CHUNK)], emb_v.at[ep],
                              sem_e[ep])
        hx = [pltpu.async_copy(x_hbm.at[pl.ds(b * (MAX_T * DM) + ts, CHUNK)],
                               x_v.at[p, b], sem_i[p])
              for b in range(NB)]
        return he, hx

    pend_in = [start_in(0)]
    pend_out = []

    for s in range(N_STEPS):
        p, ep = s % 3, s % 2
        while len(pend_out) > 1:
            for h in pend_out.pop(0):
                h.wait()
        if s + 1 < N_STEPS:
            pend_in.append(start_in(s + 1))
        he, hx = pend_in.pop(0)
        he.wait()
        for h in hx:
            h.wait()

        def grp(jg, _):
            for u in range(UNROLL):
                sl = pl.ds((jg * UNROLL + u) * LANES, LANES)
                e = emb_v[ep, sl]
                for b in range(NB):
                    plsc.addupdate(x_v.at[p, b, sl], e)
            return 0

        lax.fori_loop(0, N_VEC // UNROLL, grp, 0)

        ts = (base + s * TT) * DM
        pend_out.append([
            pltpu.async_copy(x_v.at[p, b],
                             out_hbm.at[pl.ds(b * (MAX_T * DM) + ts, CHUNK)],
                             sem_o[p])
            for b in range(NB)
        ])

    for hs in pend_out:
        for h in hs:
            h.wait()


@jax.jit
def _sc_kernel(x_flat, emb_flat):
    mesh = plsc.VectorSubcoreMesh(core_axis_name="c", subcore_axis_name="s")
    return pl.kernel(
        _sc_add,
        mesh=mesh,
        out_type=jax.ShapeDtypeStruct((NB * MAX_T * DM,), jnp.float32),
        scratch_types=[
            pltpu.VMEM((2, CHUNK), jnp.float32),
            pltpu.VMEM((3, NB, CHUNK), jnp.float32),
        ] + [pltpu.SemaphoreType.DMA] * 8,
    )(x_flat, emb_flat)


def kernel(x, emb):
    b, t, d = x.shape
    out = _sc_kernel(x.reshape(-1), emb.reshape(-1))
    return out.reshape(b, t, d)


# speedup vs baseline: 1.7071x; 1.0919x over previous
import jax
import jax.numpy as jnp
from jax import lax
from jax.experimental import pallas as pl
from jax.experimental.pallas import tpu as pltpu
from jax.experimental.pallas import tpu_sc as plsc

MAX_T = 8192
DM = 1024
NB = 4

NC = 2
NS = 16
NW = NC * NS

TT = 8
T_PER_W = MAX_T // NW
N_STEPS = T_PER_W // TT
CHUNK = TT * DM
LANES = 16
UNROLL = 4
N_VEC = CHUNK // LANES


def _sc_add(x_hbm, emb_hbm, out_hbm, emb_v, x_v,
            sem_e0, sem_e1, sem_i0, sem_i1, sem_i2,
            sem_o0, sem_o1, sem_o2):
    wid = lax.axis_index("s") * NC + lax.axis_index("c")
    base = wid * T_PER_W

    sem_e = (sem_e0, sem_e1)
    sem_i = (sem_i0, sem_i1, sem_i2)
    sem_o = (sem_o0, sem_o1, sem_o2)

    def start_in(s):
        p, ep = s % 3, s % 2
        ts = (base + s * TT) * DM
        he = pltpu.async_copy(emb_hbm.at[pl.ds(ts, CHUNK)], emb_v.at[ep],
                              sem_e[ep])
        hx = [pltpu.async_copy(x_hbm.at[pl.ds(b * (MAX_T * DM) + ts, CHUNK)],
                               x_v.at[p, b], sem_i[p])
              for b in range(NB)]
        return he, hx

    pend_in = [start_in(0)]
    pend_out = []

    for s in range(N_STEPS):
        p, ep = s % 3, s % 2
        while len(pend_out) > 1:
            for h in pend_out.pop(0):
                h.wait()
        if s + 1 < N_STEPS:
            pend_in.append(start_in(s + 1))
        he, hx = pend_in.pop(0)
        he.wait()
        for h in hx:
            h.wait()

        @plsc.parallel_loop(0, N_VEC, unroll=UNROLL)
        def _(j):
            sl = pl.ds(j * LANES, LANES)
            e = emb_v[ep, sl]
            for b in range(NB):
                plsc.addupdate(x_v.at[p, b, sl], e)

        ts = (base + s * TT) * DM
        pend_out.append([
            pltpu.async_copy(x_v.at[p, b],
                             out_hbm.at[pl.ds(b * (MAX_T * DM) + ts, CHUNK)],
                             sem_o[p])
            for b in range(NB)
        ])

    for hs in pend_out:
        for h in hs:
            h.wait()


@jax.jit
def _sc_kernel(x_flat, emb_flat):
    mesh = plsc.VectorSubcoreMesh(core_axis_name="c", subcore_axis_name="s")
    return pl.kernel(
        _sc_add,
        mesh=mesh,
        out_type=jax.ShapeDtypeStruct((NB * MAX_T * DM,), jnp.float32),
        scratch_types=[
            pltpu.VMEM((2, CHUNK), jnp.float32),
            pltpu.VMEM((3, NB, CHUNK), jnp.float32),
        ] + [pltpu.SemaphoreType.DMA] * 8,
    )(x_flat, emb_flat)


def kernel(x, emb):
    b, t, d = x.shape
    out = _sc_kernel(x.reshape(-1), emb.reshape(-1))
    return out.reshape(b, t, d)


# speedup vs baseline: 5.6407x; 3.3043x over previous
import jax
import jax.numpy as jnp
from jax import lax
from jax.experimental import pallas as pl
from jax.experimental.pallas import tpu as pltpu
from jax.experimental.pallas import tpu_sc as plsc

MAX_T = 8192
DM = 1024
NB = 4

NC = 2
NS = 16
NW = NC * NS

TT = 8
T_PER_W = MAX_T // NW
N_STEPS = T_PER_W // TT
LANES = 16
VEC_PER_ROW = DM // LANES
N_VEC = TT * VEC_PER_ROW
UNROLL = 4


def _sc_add(x_hbm, emb_hbm, out_hbm, emb_v, x_v,
            sem_e0, sem_e1, sem_i0, sem_i1, sem_i2,
            sem_o0, sem_o1, sem_o2):
    wid = lax.axis_index("s") * NC + lax.axis_index("c")
    base = wid * T_PER_W

    sem_e = (sem_e0, sem_e1)
    sem_i = (sem_i0, sem_i1, sem_i2)
    sem_o = (sem_o0, sem_o1, sem_o2)

    def start_in(s):
        p, ep = s % 3, s % 2
        ts = base + s * TT
        he = pltpu.async_copy(emb_hbm.at[pl.ds(ts, TT)], emb_v.at[ep],
                              sem_e[ep])
        hx = [pltpu.async_copy(x_hbm.at[pl.ds(b * MAX_T + ts, TT)],
                               x_v.at[p, b], sem_i[p])
              for b in range(NB)]
        return he, hx

    pend_in = [start_in(0)]
    pend_out = []

    for s in range(N_STEPS):
        p, ep = s % 3, s % 2
        while len(pend_out) > 1:
            for h in pend_out.pop(0):
                h.wait()
        if s + 1 < N_STEPS:
            pend_in.append(start_in(s + 1))
        he, hx = pend_in.pop(0)
        he.wait()
        for h in hx:
            h.wait()

        @plsc.parallel_loop(0, N_VEC, unroll=UNROLL)
        def _(j):
            r = j // VEC_PER_ROW
            sl = pl.ds((j % VEC_PER_ROW) * LANES, LANES)
            e = emb_v[ep, r, sl]
            for b in range(NB):
                plsc.addupdate(x_v.at[p, b, r, sl], e)

        ts = base + s * TT
        pend_out.append([
            pltpu.async_copy(x_v.at[p, b],
                             out_hbm.at[pl.ds(b * MAX_T + ts, TT)],
                             sem_o[p])
            for b in range(NB)
        ])

    for hs in pend_out:
        for h in hs:
            h.wait()


@jax.jit
def _sc_kernel(x2, emb):
    mesh = plsc.VectorSubcoreMesh(core_axis_name="c", subcore_axis_name="s")
    return pl.kernel(
        _sc_add,
        mesh=mesh,
        out_type=jax.ShapeDtypeStruct((NB * MAX_T, DM), jnp.float32),
        scratch_types=[
            pltpu.VMEM((2, TT, DM), jnp.float32),
            pltpu.VMEM((3, NB, TT, DM), jnp.float32),
        ] + [pltpu.SemaphoreType.DMA] * 8,
        compiler_params=pltpu.CompilerParams(use_tc_tiling_on_sc=True),
    )(x2, emb)


def kernel(x, emb):
    b, t, d = x.shape
    out = _sc_kernel(x.reshape(b * t, d), emb)
    return out.reshape(b, t, d)


# speedup vs baseline: 5.6412x; 1.0001x over previous
import jax
import jax.numpy as jnp
from jax import lax
from jax.experimental import pallas as pl
from jax.experimental.pallas import tpu as pltpu
from jax.experimental.pallas import tpu_sc as plsc

MAX_T = 8192
DM = 1024
NB = 4

NC = 2
NS = 16
NW = NC * NS

TT = 8
T_PER_W = MAX_T // NW
N_STEPS = T_PER_W // TT
LANES = 16
VEC_PER_ROW = DM // LANES
N_VEC = TT * VEC_PER_ROW
UNROLL = 8


def _sc_add(x_hbm, emb_hbm, out_hbm, emb_v, x_v,
            sem_e0, sem_e1, sem_i0, sem_i1, sem_i2,
            sem_o0, sem_o1, sem_o2):
    wid = lax.axis_index("s") * NC + lax.axis_index("c")
    base = wid * T_PER_W

    sem_e = (sem_e0, sem_e1)
    sem_i = (sem_i0, sem_i1, sem_i2)
    sem_o = (sem_o0, sem_o1, sem_o2)

    def start_in(s):
        p, ep = s % 3, s % 2
        ts = base + s * TT
        he = pltpu.async_copy(emb_hbm.at[pl.ds(ts, TT)], emb_v.at[ep],
                              sem_e[ep])
        hx = pltpu.async_copy(x_hbm.at[:, pl.ds(ts, TT)], x_v.at[p],
                              sem_i[p])
        return he, hx

    pend_in = [start_in(0)]
    pend_out = []

    for s in range(N_STEPS):
        p, ep = s % 3, s % 2
        while len(pend_out) > 1:
            pend_out.pop(0).wait()
        if s + 1 < N_STEPS:
            pend_in.append(start_in(s + 1))
        he, hx = pend_in.pop(0)
        he.wait()
        hx.wait()

        @plsc.parallel_loop(0, N_VEC, unroll=UNROLL)
        def _(j):
            r = j // VEC_PER_ROW
            sl = pl.ds((j % VEC_PER_ROW) * LANES, LANES)
            e = emb_v[ep, r, sl]
            for b in range(NB):
                plsc.addupdate(x_v.at[p, b, r, sl], e)

        ts = base + s * TT
        pend_out.append(
            pltpu.async_copy(x_v.at[p], out_hbm.at[:, pl.ds(ts, TT)],
                             sem_o[p]))

    for h in pend_out:
        h.wait()


@jax.jit
def _sc_kernel(x, emb):
    mesh = plsc.VectorSubcoreMesh(core_axis_name="c", subcore_axis_name="s")
    return pl.kernel(
        _sc_add,
        mesh=mesh,
        out_type=jax.ShapeDtypeStruct((NB, MAX_T, DM), jnp.float32),
        scratch_types=[
            pltpu.VMEM((2, TT, DM), jnp.float32),
            pltpu.VMEM((3, NB, TT, DM), jnp.float32),
        ] + [pltpu.SemaphoreType.DMA] * 8,
        compiler_params=pltpu.CompilerParams(use_tc_tiling_on_sc=True),
    )(x, emb)


def kernel(x, emb):
    return _sc_kernel(x, emb)
